# baseline (speedup 1.0000x reference)
import functools

import jax
import jax.numpy as jnp
from jax import lax
from jax.experimental import pallas as pl
from jax.experimental.pallas import tpu as pltpu
from jax.experimental.pallas import tpu_sc as plsc

VOCAB = 1_000_000
EMBED_DIM = 32
BATCH = 16384

_NC = 2
_NS = 16
_NW = _NC * _NS
_B_PER_W = BATCH // _NW

_mesh = plsc.VectorSubcoreMesh(core_axis_name="c", subcore_axis_name="s")


@functools.partial(
    pl.kernel,
    mesh=_mesh,
    out_type=jax.ShapeDtypeStruct((BATCH, EMBED_DIM), jnp.float32),
    scratch_types=[
        pltpu.VMEM((_B_PER_W,), jnp.int32),
        pltpu.SemaphoreType.DMA,
    ],
)
def _gather_kernel(idx_hbm, table_hbm, out_hbm, idx_v, sem):
    wid = lax.axis_index("s") * _NC + lax.axis_index("c")
    base = wid * _B_PER_W
    pltpu.sync_copy(idx_hbm.at[pl.ds(base, _B_PER_W)], idx_v)

    def fire(g, carry):
        idxv = idx_v[pl.ds(g * 16, 16)]
        for k in range(16):
            pltpu.make_async_copy(
                table_hbm.at[pl.ds(idxv[k], 1)],
                out_hbm.at[pl.ds(base + g * 16 + k, 1)],
                sem,
            ).start()
        return carry

    lax.fori_loop(0, _B_PER_W // 16, fire, 0)
    pltpu.make_async_copy(
        table_hbm.at[pl.ds(0, _B_PER_W)],
        out_hbm.at[pl.ds(base, _B_PER_W)],
        sem,
    ).wait()


def kernel(indices, table):
    return _gather_kernel(indices.astype(jnp.int32), table)


# speedup vs baseline: 1.3899x; 1.3899x over previous
import functools

import jax
import jax.numpy as jnp
from jax import lax
from jax.experimental import pallas as pl
from jax.experimental.pallas import tpu as pltpu
from jax.experimental.pallas import tpu_sc as plsc

VOCAB = 1_000_000
EMBED_DIM = 32
BATCH = 16384

_NC = 2
_NS = 16
_NW = _NC * _NS
_B_PER_W = BATCH // _NW
_GROUP = 64
_G = _B_PER_W // _GROUP
_T = 40

_mesh = plsc.VectorSubcoreMesh(core_axis_name="c", subcore_axis_name="s")


@functools.partial(
    pl.kernel,
    mesh=_mesh,
    out_type=jax.ShapeDtypeStruct((BATCH, EMBED_DIM), jnp.float32),
    scratch_types=[
        pltpu.VMEM((_B_PER_W,), jnp.int32),
        pltpu.VMEM((_G, _T, EMBED_DIM), jnp.float32),
        pltpu.SemaphoreType.DMA,
        pltpu.SemaphoreType.DMA,
        pltpu.SemaphoreType.DMA,
    ],
)
def _gather_kernel(idx_hbm, table_hbm, out_hbm, idx_v, rows_v, sem_s, sem_d, sem_w):
    wid = lax.axis_index("s") * _NC + lax.axis_index("c")
    base = wid * _B_PER_W
    pltpu.sync_copy(idx_hbm.at[pl.ds(base, _B_PER_W)], idx_v)

    def fire(g, carry):
        for q in range(_GROUP // 16):
            idxv = idx_v[pl.ds(g * _GROUP + q * 16, 16)]
            for k in range(16):
                r = q * 16 + k
                if r < _T:
                    pltpu.make_async_copy(
                        table_hbm.at[pl.ds(idxv[k], 1)],
                        rows_v.at[g, pl.ds(r, 1)],
                        sem_s,
                    ).start()
                else:
                    pltpu.make_async_copy(
                        table_hbm.at[pl.ds(idxv[k], 1)],
                        out_hbm.at[pl.ds(base + g * _GROUP + r, 1)],
                        sem_d,
                    ).start()
        return carry

    lax.fori_loop(0, _G, fire, 0)

    def drain_and_writeback(g, carry):
        pltpu.make_async_copy(
            table_hbm.at[pl.ds(0, _T)], rows_v.at[g], sem_s
        ).wait()
        pltpu.make_async_copy(
            rows_v.at[g], out_hbm.at[pl.ds(base + g * _GROUP, _T)], sem_w
        ).start()
        return carry

    lax.fori_loop(0, _G, drain_and_writeback, 0)

    def drain_wb(g, carry):
        pltpu.make_async_copy(
            rows_v.at[g], out_hbm.at[pl.ds(base + g * _GROUP, _T)], sem_w
        ).wait()
        return carry

    lax.fori_loop(0, _G, drain_wb, 0)

    def drain_direct(g, carry):
        pltpu.make_async_copy(
            table_hbm.at[pl.ds(0, _GROUP - _T)],
            out_hbm.at[pl.ds(base + g * _GROUP + _T, _GROUP - _T)],
            sem_d,
        ).wait()
        return carry

    lax.fori_loop(0, _G, drain_direct, 0)


def kernel(indices, table):
    return _gather_kernel(indices.astype(jnp.int32), table)


# speedup vs baseline: 1.7918x; 1.2892x over previous
import functools

import jax
import jax.numpy as jnp
from jax import lax
from jax.experimental import pallas as pl
from jax.experimental.pallas import tpu as pltpu
from jax.experimental.pallas import tpu_sc as plsc

VOCAB = 1_000_000
EMBED_DIM = 32
BATCH = 16384

_NC = 2
_NS = 16
_NW = _NC * _NS
_B_PER_W = BATCH // _NW

_mesh = plsc.VectorSubcoreMesh(core_axis_name="c", subcore_axis_name="s")


@functools.partial(
    pl.kernel,
    mesh=_mesh,
    out_type=jax.ShapeDtypeStruct((BATCH, EMBED_DIM), jnp.float32),
    scratch_types=[
        pltpu.VMEM((_B_PER_W,), jnp.int32),
        pltpu.VMEM((_B_PER_W, EMBED_DIM), jnp.float32),
        pltpu.SemaphoreType.DMA,
    ],
)
def _gather_kernel(idx_hbm, table_hbm, out_hbm, idx_v, rows_v, sem):
    wid = lax.axis_index("s") * _NC + lax.axis_index("c")
    base = wid * _B_PER_W
    pltpu.sync_copy(idx_hbm.at[pl.ds(base, _B_PER_W)], idx_v)

    def fire(g, carry):
        idxv = idx_v[pl.ds(g * 16, 16)]
        for k in range(16):
            pltpu.make_async_copy(
                table_hbm.at[pl.ds(idxv[k], 1)],
                rows_v.at[pl.ds(g * 16 + k, 1)],
                sem,
            ).start()
        return carry

    lax.fori_loop(0, _B_PER_W // 16, fire, 0)
    pltpu.make_async_copy(
        table_hbm.at[pl.ds(0, _B_PER_W)],
        rows_v,
        sem,
    ).wait()

    pltpu.sync_copy(rows_v, out_hbm.at[pl.ds(base, _B_PER_W)])


def kernel(indices, table):
    return _gather_kernel(indices.astype(jnp.int32), table)
